# baseline (speedup 1.0000x reference)
import functools

import jax
import jax.numpy as jnp
from jax import lax
from jax.experimental import pallas as pl
from jax.experimental.pallas import tpu as pltpu
from jax.experimental.pallas import tpu_sc as plsc

SIGMA = 1.0

_B = 1024
_N = 200
_NC = 2
_NS = 16
_NW = _NC * _NS
_RPW = _B // _NW
_FPW = _RPW * _N
_NCHUNK = 13
_NP = _NCHUNK * 16
_INF = float("inf")


def _bcast_lane(vec, idx16):
    dnums = lax.GatherDimensionNumbers(
        offset_dims=(), collapsed_slice_dims=(0,), start_index_map=(0,))
    return lax.gather(vec, idx16[:, None], dnums, (1,),
                      mode=lax.GatherScatterMode.PROMISE_IN_BOUNDS)


def _sc_body(sims_hbm, levs_hbm, out_hbm, ss_v, sl_v, d_v, l_v, o_v):
    wid = lax.axis_index("s") * _NC + lax.axis_index("c")
    base = wid * _FPW

    pltpu.sync_copy(sims_hbm.at[pl.ds(base, _FPW)], ss_v.at[pl.ds(0, _FPW)])
    pltpu.sync_copy(levs_hbm.at[pl.ds(base, _FPW)], sl_v.at[pl.ds(0, _FPW)])

    iota = lax.iota(jnp.int32, 16)
    head8 = iota < 8

    def lay_row(r, carry):
        src = r * _N
        dst = r * _NP
        for c in range(12):
            sv = ss_v[pl.ds(src + 16 * c, 16)]
            lv = sl_v[pl.ds(src + 16 * c, 16)]
            d_v[pl.ds(dst + 16 * c, 16)] = sv - SIGMA * lv
            l_v[pl.ds(dst + 16 * c, 16)] = lv
        sv = ss_v[pl.ds(src + 192, 16)]
        lv = sl_v[pl.ds(src + 192, 16)]
        d_v[pl.ds(dst + 192, 16)] = jnp.where(head8, sv - SIGMA * lv, _INF)
        l_v[pl.ds(dst + 192, 16)] = jnp.where(head8, lv, _INF)
        return carry
    lax.fori_loop(0, _RPW, lay_row, 0)

    zero16 = jnp.zeros((16,), jnp.float32)

    def row_body(r, accs):
        rbase = r * _NP
        dqs = [d_v[pl.ds(rbase + 16 * c, 16)] for c in range(_NCHUNK)]
        lqs = [l_v[pl.ds(rbase + 16 * c, 16)] for c in range(_NCHUNK)]

        for cp in range(_NCHUNK):
            def i_body(i, accs, cp=cp):
                ib = jnp.full((16,), i, jnp.int32)
                dp = _bcast_lane(dqs[cp], ib)
                lp = _bcast_lane(lqs[cp], ib)
                qmask = iota > ib
                new = list(accs)
                t = dp - dqs[cp]
                v = jnp.where(lp <= lqs[cp], t, -t)
                c = jnp.maximum(v, 0.0)
                new[cp] = new[cp] + jnp.where(qmask, c, zero16)
                for cq in range(cp + 1, _NCHUNK):
                    t = dp - dqs[cq]
                    v = jnp.where(lp <= lqs[cq], t, -t)
                    new[cq] = new[cq] + jnp.maximum(v, 0.0)
                return tuple(new)
            accs = lax.fori_loop(0, 16, i_body, accs)
        return accs

    accs = lax.fori_loop(0, _RPW, row_body, (zero16,) * _NCHUNK)
    total = accs[0]
    for c in range(1, _NCHUNK):
        total = total + accs[c]
    o_v[...] = total
    pltpu.sync_copy(o_v, out_hbm.at[wid])


@jax.jit
def _sc_pairwise(similarities, levs):
    mesh = plsc.VectorSubcoreMesh(core_axis_name="c", subcore_axis_name="s")
    f = functools.partial(
        pl.kernel,
        out_type=jax.ShapeDtypeStruct((_NW, 16), jnp.float32),
        mesh=mesh,
        scratch_types=[
            pltpu.VMEM((_FPW + 16,), jnp.float32),
            pltpu.VMEM((_FPW + 16,), jnp.float32),
            pltpu.VMEM((_RPW * _NP + 16,), jnp.float32),
            pltpu.VMEM((_RPW * _NP + 16,), jnp.float32),
            pltpu.VMEM((16,), jnp.float32),
        ],
    )(_sc_body)
    return f(similarities.reshape(-1), levs.reshape(-1))


def kernel(similarities, levs):
    levs = levs.reshape(similarities.shape)
    partials = _sc_pairwise(similarities, levs)
    return jnp.sum(partials) / jnp.float32(_B * _N * _N)


# speedup vs baseline: 1.1325x; 1.1325x over previous
import functools

import jax
import jax.numpy as jnp
from jax import lax
from jax.experimental import pallas as pl
from jax.experimental.pallas import tpu as pltpu
from jax.experimental.pallas import tpu_sc as plsc

SIGMA = 1.0

_B = 1024
_N = 200
_NC = 2
_NS = 16
_NW = _NC * _NS
_SC_ROWS = 512
_RPW = _SC_ROWS // _NW
_FPW = _RPW * _N
_NCHUNK = 13
_NP = _NCHUNK * 16
_INF = float("inf")
_TC_RB = 8


def _sc_body(sims_hbm, levs_hbm, out_hbm, ss_v, sl_v, d_v, l_v, o_v):
    wid = lax.axis_index("s") * _NC + lax.axis_index("c")
    base = wid * _FPW

    pltpu.sync_copy(sims_hbm.at[pl.ds(base, _FPW)], ss_v.at[pl.ds(0, _FPW)])
    pltpu.sync_copy(levs_hbm.at[pl.ds(base, _FPW)], sl_v.at[pl.ds(0, _FPW)])

    iota = lax.iota(jnp.int32, 16)
    head8 = iota < 8

    def lay_row(r, carry):
        src = r * _N
        dst = r * _NP
        for c in range(12):
            sv = ss_v[pl.ds(src + 16 * c, 16)]
            lv = sl_v[pl.ds(src + 16 * c, 16)]
            d_v[pl.ds(dst + 16 * c, 16)] = sv - SIGMA * lv
            l_v[pl.ds(dst + 16 * c, 16)] = lv
        sv = ss_v[pl.ds(src + 192, 16)]
        lv = sl_v[pl.ds(src + 192, 16)]
        d_v[pl.ds(dst + 192, 16)] = jnp.where(head8, sv - SIGMA * lv, _INF)
        l_v[pl.ds(dst + 192, 16)] = jnp.where(head8, lv, _INF)
        return carry
    lax.fori_loop(0, _RPW, lay_row, 0)

    zero16 = jnp.zeros((16,), jnp.float32)

    def row_body(r, accs):
        rbase = r * _NP
        dqs = [d_v[pl.ds(rbase + 16 * c, 16)] for c in range(_NCHUNK)]
        lqs = [l_v[pl.ds(rbase + 16 * c, 16)] for c in range(_NCHUNK)]

        for cp in range(_NCHUNK):
            def i_body(i, accs, cp=cp):
                pa = rbase + 16 * cp + i
                dp = jnp.full((16,), d_v[pl.ds(pa, 16)][0], jnp.float32)
                lp = jnp.full((16,), l_v[pl.ds(pa, 16)][0], jnp.float32)
                qmask = iota > jnp.full((16,), i, jnp.int32)
                new = list(accs)
                t = dp - dqs[cp]
                v = jnp.where(lp <= lqs[cp], t, -t)
                c = jnp.maximum(v, 0.0)
                new[cp] = new[cp] + jnp.where(qmask, c, zero16)
                for cq in range(cp + 1, _NCHUNK):
                    t = dp - dqs[cq]
                    v = jnp.where(lp <= lqs[cq], t, -t)
                    new[cq] = new[cq] + jnp.maximum(v, 0.0)
                return tuple(new)
            accs = lax.fori_loop(0, 16, i_body, accs)
        return accs

    accs = lax.fori_loop(0, _RPW, row_body, (zero16,) * _NCHUNK)
    total = accs[0]
    for c in range(1, _NCHUNK):
        total = total + accs[c]
    o_v[...] = total
    pltpu.sync_copy(o_v, out_hbm.at[wid])


@jax.jit
def _sc_pairwise(sims_flat, levs_flat):
    mesh = plsc.VectorSubcoreMesh(core_axis_name="c", subcore_axis_name="s")
    f = functools.partial(
        pl.kernel,
        out_type=jax.ShapeDtypeStruct((_NW, 16), jnp.float32),
        mesh=mesh,
        scratch_types=[
            pltpu.VMEM((_FPW + 16,), jnp.float32),
            pltpu.VMEM((_FPW + 16,), jnp.float32),
            pltpu.VMEM((_RPW * _NP + 16,), jnp.float32),
            pltpu.VMEM((_RPW * _NP + 16,), jnp.float32),
            pltpu.VMEM((16,), jnp.float32),
        ],
    )(_sc_body)
    return f(sims_flat, levs_flat)


def _tc_body(s_ref, l_ref, out_ref):
    s = s_ref[...]
    l = l_ref[...]
    d = s - SIGMA * l
    colq = lax.broadcasted_iota(jnp.int32, (_TC_RB, _N), 1)
    zero = jnp.zeros((_TC_RB, _N), jnp.float32)

    acc = zero
    for p in range(_N):
        dp = jnp.broadcast_to(d[:, p : p + 1], (_TC_RB, _N))
        lp = jnp.broadcast_to(l[:, p : p + 1], (_TC_RB, _N))
        t = dp - d
        v = jnp.where(lp <= l, t, -t)
        c = jnp.maximum(v, 0.0)
        acc = acc + jnp.where(colq > p, c, zero)
    out_ref[...] = jnp.sum(acc, axis=0, keepdims=True)[None]


@jax.jit
def _tc_pairwise(similarities, levs):
    nblk = (_B - _SC_ROWS) // _TC_RB
    blk0 = _SC_ROWS // _TC_RB
    return pl.pallas_call(
        _tc_body,
        out_shape=jax.ShapeDtypeStruct((nblk, 1, _N), jnp.float32),
        grid=(nblk,),
        in_specs=[
            pl.BlockSpec((_TC_RB, _N), lambda b: (blk0 + b, 0)),
            pl.BlockSpec((_TC_RB, _N), lambda b: (blk0 + b, 0)),
        ],
        out_specs=pl.BlockSpec((1, 1, _N), lambda b: (b, 0, 0)),
    )(similarities, levs)


def kernel(similarities, levs):
    levs = levs.reshape(similarities.shape)
    sc_part = _sc_pairwise(
        similarities.reshape(-1)[: _SC_ROWS * _N],
        levs.reshape(-1)[: _SC_ROWS * _N],
    )
    tc_part = _tc_pairwise(similarities, levs)
    total = jnp.sum(sc_part) + jnp.sum(tc_part)
    return total / jnp.float32(_B * _N * _N)


# speedup vs baseline: 1.3226x; 1.1679x over previous
import functools

import jax
import jax.numpy as jnp
from jax import lax
from jax.experimental import pallas as pl
from jax.experimental.pallas import tpu as pltpu
from jax.experimental.pallas import tpu_sc as plsc

SIGMA = 1.0

_B = 1024
_N = 200
_NC = 2
_NS = 16
_NW = _NC * _NS
_SC_ROWS = 608
_RPW = _SC_ROWS // _NW
_FPW = _RPW * _N
_NCHUNK = 13
_NP = _NCHUNK * 16
_INF = float("inf")
_TC_RB = 8


def _sc_body(sims_hbm, levs_hbm, out_hbm, ss_v, sl_v, d_v, l_v, o_v):
    wid = lax.axis_index("s") * _NC + lax.axis_index("c")
    base = wid * _FPW

    pltpu.sync_copy(sims_hbm.at[pl.ds(base, _FPW)], ss_v.at[pl.ds(0, _FPW)])
    pltpu.sync_copy(levs_hbm.at[pl.ds(base, _FPW)], sl_v.at[pl.ds(0, _FPW)])

    iota = lax.iota(jnp.int32, 16)
    head8 = iota < 8

    def lay_row(r, carry):
        src = r * _N
        dst = r * _NP
        for c in range(12):
            sv = ss_v[pl.ds(src + 16 * c, 16)]
            lv = sl_v[pl.ds(src + 16 * c, 16)]
            d_v[pl.ds(dst + 16 * c, 16)] = sv - SIGMA * lv
            l_v[pl.ds(dst + 16 * c, 16)] = lv
        sv = ss_v[pl.ds(src + 192, 16)]
        lv = sl_v[pl.ds(src + 192, 16)]
        d_v[pl.ds(dst + 192, 16)] = jnp.where(head8, sv - SIGMA * lv, _INF)
        l_v[pl.ds(dst + 192, 16)] = jnp.where(head8, lv, _INF)
        return carry
    lax.fori_loop(0, _RPW, lay_row, 0)

    zero16 = jnp.zeros((16,), jnp.float32)

    def row_body(r, accs):
        rbase = r * _NP
        dqs = [d_v[pl.ds(rbase + 16 * c, 16)] for c in range(_NCHUNK)]
        lqs = [l_v[pl.ds(rbase + 16 * c, 16)] for c in range(_NCHUNK)]

        for cp in range(_NCHUNK):
            def i_body(i, accs, cp=cp):
                pa = rbase + 16 * cp + i
                dp = jnp.full((16,), d_v[pl.ds(pa, 16)][0], jnp.float32)
                lp = jnp.full((16,), l_v[pl.ds(pa, 16)][0], jnp.float32)
                qmask = iota > jnp.full((16,), i, jnp.int32)
                new = list(accs)
                t = dp - dqs[cp]
                v = jnp.where(lp <= lqs[cp], t, -t)
                c = jnp.maximum(v, 0.0)
                new[cp] = new[cp] + jnp.where(qmask, c, zero16)
                for cq in range(cp + 1, _NCHUNK):
                    t = dp - dqs[cq]
                    v = jnp.where(lp <= lqs[cq], t, -t)
                    new[cq] = new[cq] + jnp.maximum(v, 0.0)
                return tuple(new)
            accs = lax.fori_loop(0, 16, i_body, accs)
        return accs

    accs = lax.fori_loop(0, _RPW, row_body, (zero16,) * _NCHUNK)
    total = accs[0]
    for c in range(1, _NCHUNK):
        total = total + accs[c]
    o_v[...] = total
    pltpu.sync_copy(o_v, out_hbm.at[wid])


@jax.jit
def _sc_pairwise(sims_flat, levs_flat):
    mesh = plsc.VectorSubcoreMesh(core_axis_name="c", subcore_axis_name="s")
    f = functools.partial(
        pl.kernel,
        out_type=jax.ShapeDtypeStruct((_NW, 16), jnp.float32),
        mesh=mesh,
        scratch_types=[
            pltpu.VMEM((_FPW + 16,), jnp.float32),
            pltpu.VMEM((_FPW + 16,), jnp.float32),
            pltpu.VMEM((_RPW * _NP + 16,), jnp.float32),
            pltpu.VMEM((_RPW * _NP + 16,), jnp.float32),
            pltpu.VMEM((16,), jnp.float32),
        ],
    )(_sc_body)
    return f(sims_flat, levs_flat)


def _tc_body(s_ref, l_ref, out_ref):
    s = s_ref[...]
    l = l_ref[...]
    d = s - SIGMA * l
    colq = lax.broadcasted_iota(jnp.int32, (_TC_RB, _N), 1)
    zero = jnp.zeros((_TC_RB, _N), jnp.float32)

    acc = zero
    for p in range(128):
        dp = jnp.broadcast_to(d[:, p : p + 1], (_TC_RB, _N))
        lp = jnp.broadcast_to(l[:, p : p + 1], (_TC_RB, _N))
        t = dp - d
        v = jnp.where(lp <= l, t, -t)
        c = jnp.maximum(v, 0.0)
        acc = acc + jnp.where(colq > p, c, zero)

    nhi = _N - 128
    dh = d[:, 128:]
    lh = l[:, 128:]
    colqh = colq[:, 128:]
    zeroh = jnp.zeros((_TC_RB, nhi), jnp.float32)
    acch = zeroh
    for p in range(128, _N):
        dp = jnp.broadcast_to(d[:, p : p + 1], (_TC_RB, nhi))
        lp = jnp.broadcast_to(l[:, p : p + 1], (_TC_RB, nhi))
        t = dp - dh
        v = jnp.where(lp <= lh, t, -t)
        c = jnp.maximum(v, 0.0)
        acch = acch + jnp.where(colqh > p, c, zeroh)

    row_tot = jnp.sum(acc, axis=0, keepdims=True)
    row_hi = jnp.sum(acch, axis=0, keepdims=True)
    row_tot = row_tot + jnp.concatenate(
        [jnp.zeros((1, 128), jnp.float32), row_hi], axis=1)
    out_ref[...] = row_tot[None]


@jax.jit
def _tc_pairwise(similarities, levs):
    nblk = (_B - _SC_ROWS) // _TC_RB
    blk0 = _SC_ROWS // _TC_RB
    return pl.pallas_call(
        _tc_body,
        out_shape=jax.ShapeDtypeStruct((nblk, 1, _N), jnp.float32),
        grid=(nblk,),
        in_specs=[
            pl.BlockSpec((_TC_RB, _N), lambda b: (blk0 + b, 0)),
            pl.BlockSpec((_TC_RB, _N), lambda b: (blk0 + b, 0)),
        ],
        out_specs=pl.BlockSpec((1, 1, _N), lambda b: (b, 0, 0)),
    )(similarities, levs)


def kernel(similarities, levs):
    levs = levs.reshape(similarities.shape)
    sc_part = _sc_pairwise(similarities.reshape(-1), levs.reshape(-1))
    tc_part = _tc_pairwise(similarities, levs)
    total = jnp.sum(sc_part) + jnp.sum(tc_part)
    return total / jnp.float32(_B * _N * _N)
